# initial kernel scaffold (unmeasured)
import jax
import jax.numpy as jnp
from jax import lax
from jax.experimental import pallas as pl
from jax.experimental.pallas import tpu as pltpu


def kernel(
    x,
):
    def body(*refs):
        pass

    out_shape = jax.ShapeDtypeStruct(..., jnp.float32)
    return pl.pallas_call(body, out_shape=out_shape)(...)


# baseline (device time: 9269 ns/iter reference)
import jax
import jax.numpy as jnp
from jax import lax
from jax.experimental import pallas as pl
from jax.experimental.pallas import tpu as pltpu

N_DEV = 4


def kernel(x):
    m, n = x.shape

    def body(x_ref, out_ref, comm_ref, send_ref, send_sem, recv_sem):
        my = lax.axis_index("i")
        left = (my - 1) % N_DEV
        right = (my + 1) % N_DEV

        comm_ref[...] = jnp.ones((1, n), jnp.float32)

        barrier_sem = pltpu.get_barrier_semaphore()
        for nbr in [left, right]:
            pl.semaphore_signal(
                barrier_sem, inc=1,
                device_id=(nbr,), device_id_type=pl.DeviceIdType.MESH,
            )
        pl.semaphore_wait(barrier_sem, 2)

        local = x_ref[...]
        d = 1
        while d < m:
            shifted = jnp.concatenate(
                [jnp.ones((d, n), jnp.float32), local[: m - d, :]], axis=0
            )
            local = local * shifted
            d *= 2

        @pl.when(my > 0)
        def _():
            recv = pltpu.make_async_remote_copy(
                src_ref=send_ref, dst_ref=comm_ref,
                send_sem=send_sem, recv_sem=recv_sem,
                device_id=(left,), device_id_type=pl.DeviceIdType.MESH,
            )
            recv.wait_recv()

        prefix = comm_ref[...]
        out = local * prefix
        out_ref[...] = out
        send_ref[...] = out[m - 1:m, :]

        @pl.when(my < N_DEV - 1)
        def _():
            send = pltpu.make_async_remote_copy(
                src_ref=send_ref, dst_ref=comm_ref,
                send_sem=send_sem, recv_sem=recv_sem,
                device_id=(right,), device_id_type=pl.DeviceIdType.MESH,
            )
            send.start()
            send.wait_send()

    return pl.pallas_call(
        body,
        out_shape=jax.ShapeDtypeStruct((m, n), jnp.float32),
        in_specs=[pl.BlockSpec(memory_space=pltpu.VMEM)],
        out_specs=pl.BlockSpec(memory_space=pltpu.VMEM),
        scratch_shapes=[
            pltpu.VMEM((1, n), jnp.float32),
            pltpu.VMEM((1, n), jnp.float32),
            pltpu.SemaphoreType.DMA,
            pltpu.SemaphoreType.DMA,
        ],
        compiler_params=pltpu.CompilerParams(collective_id=0),
    )(x)


# device time: 6913 ns/iter; 1.3408x vs baseline; 1.3408x over previous
import jax
import jax.numpy as jnp
from jax import lax
from jax.experimental import pallas as pl
from jax.experimental.pallas import tpu as pltpu

N_DEV = 4


def kernel(x):
    m, n = x.shape

    def body(x_ref, out_ref, comm_ref, send_ref, send_sems, recv_sems):
        my = lax.axis_index("i")

        comm_ref[...] = jnp.ones((N_DEV - 1, 1, n), jnp.float32)

        barrier_sem = pltpu.get_barrier_semaphore()
        for other in range(N_DEV):
            @pl.when(my != other)
            def _(other=other):
                pl.semaphore_signal(
                    barrier_sem, inc=1,
                    device_id=(other,), device_id_type=pl.DeviceIdType.MESH,
                )
        pl.semaphore_wait(barrier_sem, N_DEV - 1)

        tot = x_ref[...]
        half = m
        while half > 1:
            half //= 2
            tot = tot[:half, :] * tot[half:, :]
        send_ref[...] = tot

        def mk(s, r):
            return pltpu.make_async_remote_copy(
                src_ref=send_ref,
                dst_ref=comm_ref.at[s],
                send_sem=send_sems.at[r],
                recv_sem=recv_sems.at[s],
                device_id=(r,),
                device_id_type=pl.DeviceIdType.MESH,
            )

        for s in range(N_DEV - 1):
            @pl.when(my == s)
            def _(s=s):
                for r in range(s + 1, N_DEV):
                    mk(s, r).start()

        local = x_ref[...]
        d = 1
        while d < m:
            shifted = jnp.concatenate(
                [jnp.ones((d, n), jnp.float32), local[: m - d, :]], axis=0
            )
            local = local * shifted
            d *= 2

        for s in range(N_DEV - 1):
            @pl.when(my > s)
            def _(s=s):
                mk(s, (s + 1) % N_DEV).wait_recv()

        prefix = comm_ref[0] * comm_ref[1] * comm_ref[2]
        out_ref[...] = local * prefix

        for s in range(N_DEV - 1):
            @pl.when(my == s)
            def _(s=s):
                for r in range(s + 1, N_DEV):
                    mk(s, r).wait_send()

    return pl.pallas_call(
        body,
        out_shape=jax.ShapeDtypeStruct((m, n), jnp.float32),
        in_specs=[pl.BlockSpec(memory_space=pltpu.VMEM)],
        out_specs=pl.BlockSpec(memory_space=pltpu.VMEM),
        scratch_shapes=[
            pltpu.VMEM((N_DEV - 1, 1, n), jnp.float32),
            pltpu.VMEM((1, n), jnp.float32),
            pltpu.SemaphoreType.DMA((N_DEV,)),
            pltpu.SemaphoreType.DMA((N_DEV,)),
        ],
        compiler_params=pltpu.CompilerParams(collective_id=0),
    )(x)
